# baseline (speedup 1.0000x reference)
import functools

import jax
import jax.numpy as jnp
from jax import lax
from jax.experimental import pallas as pl
from jax.experimental.pallas import tpu as pltpu
from jax.experimental.pallas import tpu_sc as plsc

_N = 100000
_D1 = 256
_D2 = 512
_NC, _NS, _L = 2, 16, 16
_NW = _NC * _NS
_CH = 120
_NCH = 26
_PAIRS = _NCH // 2
_RW = _CH * _NCH
_TAIL = _N - _NW * _RW
_NTAILW = _TAIL // 8
_G1 = _D1 // _L
_G2 = _D2 // _L

_mesh = plsc.VectorSubcoreMesh(
    core_axis_name="c", subcore_axis_name="s",
    num_cores=_NC, num_subcores=_NS)


@functools.partial(
    pl.kernel,
    out_type=jax.ShapeDtypeStruct((_NW, 1, 2 * _D1), jnp.float32),
    mesh=_mesh,
    scratch_types=[
        pltpu.VMEM((2, _CH, _D1), jnp.float32),
        pltpu.VMEM((1, 2 * _D1), jnp.float32),
        pltpu.SemaphoreType.DMA,
        pltpu.SemaphoreType.DMA,
    ],
)
def _stats_kernel(md_hbm, out_hbm, buf, statbuf, sem0, sem1):
    wid = lax.axis_index("s") * _NC + lax.axis_index("c")
    base = wid * _RW

    def chunk_src(c):
        return md_hbm.at[pl.ds(base + c * _CH, _CH)]

    def make_rows2_body(b):
        def body(i, carry):
            out = list(carry)
            r0 = 2 * i
            for c in range(_G1):
                v0 = buf[b, r0, pl.ds(c * _L, _L)]
                v1 = buf[b, r0 + 1, pl.ds(c * _L, _L)]
                out[c] = (out[c] + v0) + v1
                out[_G1 + c] = (out[_G1 + c] + v0 * v0) + v1 * v1
            return tuple(out)
        return body

    def make_row_body(b):
        def body(r, carry):
            out = list(carry)
            for c in range(_G1):
                v = buf[b, r, pl.ds(c * _L, _L)]
                out[c] = out[c] + v
                out[_G1 + c] = out[_G1 + c] + v * v
            return tuple(out)
        return body

    pltpu.async_copy(chunk_src(0), buf.at[0], sem0)

    def pair_body(g, carry):
        c0 = 2 * g
        pltpu.async_copy(chunk_src(c0 + 1), buf.at[1], sem1)
        pltpu.make_async_copy(chunk_src(0), buf.at[0], sem0).wait()
        carry = lax.fori_loop(0, _CH // 2, make_rows2_body(0), carry)

        @pl.when(g < _PAIRS - 1)
        def _():
            pltpu.async_copy(chunk_src(c0 + 2), buf.at[0], sem0)

        pltpu.make_async_copy(chunk_src(0), buf.at[1], sem1).wait()
        carry = lax.fori_loop(0, _CH // 2, make_rows2_body(1), carry)
        return carry

    acc = tuple(jnp.zeros((_L,), jnp.float32) for _ in range(2 * _G1))
    acc = lax.fori_loop(0, _PAIRS, pair_body, acc)

    has_tail = wid < _NTAILW

    @pl.when(has_tail)
    def _():
        pltpu.sync_copy(md_hbm.at[pl.ds(_NW * _RW + 8 * wid, 8)],
                        buf.at[0, pl.ds(0, 8)])

    acc = lax.fori_loop(0, jnp.where(has_tail, 8, 0),
                        make_row_body(0), acc)

    for c in range(_G1):
        statbuf[0, pl.ds(c * _L, _L)] = acc[c]
        statbuf[0, pl.ds(_D1 + c * _L, _L)] = acc[_G1 + c]
    pltpu.sync_copy(statbuf, out_hbm.at[wid])


def _encoder_body(parts_ref, x_ref, w_ref, b_ref, out_ref):
    parts = parts_ref[...]
    sums = jnp.sum(parts[:, :_D1], axis=0, keepdims=True)
    sumsq = jnp.sum(parts[:, _D1:], axis=0, keepdims=True)
    mean = sums / _N
    var = jnp.maximum((sumsq - sums * mean) / (_N - 1), 0.0)
    std = jnp.sqrt(var)
    new = (x_ref[...] - mean) / std
    new = jnp.where(std == 0.0, jnp.zeros_like(new), new)
    z = jnp.dot(new, w_ref[...], preferred_element_type=jnp.float32)
    out_ref[...] = jnp.tanh(z + b_ref[...])


_encoder = pl.pallas_call(
    _encoder_body,
    out_shape=jax.ShapeDtypeStruct((1, _D2), jnp.float32),
)


@functools.partial(
    pl.kernel,
    out_type=jax.ShapeDtypeStruct((_NW, 1, _L), jnp.float32),
    mesh=_mesh,
    scratch_types=[
        pltpu.VMEM((2, _CH, _D2), jnp.float32),
        pltpu.VMEM((_D2,), jnp.float32),
        pltpu.VMEM((1, _L), jnp.float32),
        pltpu.SemaphoreType.DMA,
        pltpu.SemaphoreType.DMA,
    ],
)
def _dist_kernel(mem_hbm, e_hbm, out_hbm, buf, e_v, min_v, sem0, sem1):
    wid = lax.axis_index("s") * _NC + lax.axis_index("c")
    base = wid * _RW

    pltpu.sync_copy(e_hbm, e_v)
    evecs = [e_v[pl.ds(c * _L, _L)] for c in range(_G2)]

    iota16 = lax.iota(jnp.int32, _L)
    _dnums = lax.GatherDimensionNumbers(
        offset_dims=(), collapsed_slice_dims=(0,), start_index_map=(0,))
    perms = [(iota16 ^ k).reshape(_L, 1) for k in (1, 2, 4, 8)]

    def lane_total(v):
        for perm in perms:
            v = v + lax.gather(v, perm, _dnums, slice_sizes=(1,),
                               mode=lax.GatherScatterMode.PROMISE_IN_BOUNDS)
        return v

    def row_dist(b, r):
        a = [jnp.abs(buf[b, r, pl.ds(j * _L, _L)] - evecs[j])
             for j in range(4)]
        for c in range(4, _G2):
            a[c % 4] = a[c % 4] + jnp.abs(
                buf[b, r, pl.ds(c * _L, _L)] - evecs[c])
        return lane_total((a[0] + a[1]) + (a[2] + a[3]))

    def make_rows2_body(b):
        def body(i, m):
            r0 = 2 * i
            m = jnp.minimum(m, row_dist(b, r0))
            return jnp.minimum(m, row_dist(b, r0 + 1))
        return body

    def make_row_body(b):
        def body(r, m):
            return jnp.minimum(m, row_dist(b, r))
        return body

    def chunk_src(c):
        return mem_hbm.at[pl.ds(base + c * _CH, _CH)]

    pltpu.async_copy(chunk_src(0), buf.at[0], sem0)

    def pair_body(g, m):
        c0 = 2 * g
        pltpu.async_copy(chunk_src(c0 + 1), buf.at[1], sem1)
        pltpu.make_async_copy(chunk_src(0), buf.at[0], sem0).wait()
        m = lax.fori_loop(0, _CH // 2, make_rows2_body(0), m)

        @pl.when(g < _PAIRS - 1)
        def _():
            pltpu.async_copy(chunk_src(c0 + 2), buf.at[0], sem0)

        pltpu.make_async_copy(chunk_src(0), buf.at[1], sem1).wait()
        m = lax.fori_loop(0, _CH // 2, make_rows2_body(1), m)
        return m

    m = jnp.full((_L,), jnp.inf, jnp.float32)
    m = lax.fori_loop(0, _PAIRS, pair_body, m)

    has_tail = wid < _NTAILW

    @pl.when(has_tail)
    def _():
        pltpu.sync_copy(mem_hbm.at[pl.ds(_NW * _RW + 8 * wid, 8)],
                        buf.at[0, pl.ds(0, 8)])

    m = lax.fori_loop(0, jnp.where(has_tail, 8, 0), make_row_body(0), m)

    min_v[...] = m.reshape(1, _L)
    pltpu.sync_copy(min_v, out_hbm.at[wid])


def kernel(x, memory, mem_data, W_enc, b_enc):
    parts = _stats_kernel(mem_data)
    e = _encoder(parts.reshape(_NW, 2 * _D1), x, W_enc, b_enc.reshape(1, _D2))
    mins = _dist_kernel(memory, e.reshape(_D2))
    return jnp.min(mins)


# speedup vs baseline: 1.2670x; 1.2670x over previous
import functools

import jax
import jax.numpy as jnp
from jax import lax
from jax.experimental import pallas as pl
from jax.experimental.pallas import tpu as pltpu
from jax.experimental.pallas import tpu_sc as plsc

_N = 100000
_D1 = 256
_D2 = 512
_NC, _NS, _L = 2, 16, 16
_NW = _NC * _NS
_CH = 40
_NB = 6
_NCH = 78
_RW = _CH * _NCH
_TAIL = _N - _NW * _RW
_NTAILW = _TAIL // 8
_G1 = _D1 // _L
_G2 = _D2 // _L
_BLK = 8

_mesh = plsc.VectorSubcoreMesh(
    core_axis_name="c", subcore_axis_name="s",
    num_cores=_NC, num_subcores=_NS)


@functools.partial(
    pl.kernel,
    out_type=jax.ShapeDtypeStruct((_NW, 1, 2 * _D1), jnp.float32),
    mesh=_mesh,
    scratch_types=[
        pltpu.VMEM((_NB, _CH, _D1), jnp.float32),
        pltpu.VMEM((1, 2 * _D1), jnp.float32),
        pltpu.SemaphoreType.DMA((_NB,)),
    ],
)
def _stats_kernel(md_hbm, out_hbm, buf, statbuf, sems):
    wid = lax.axis_index("s") * _NC + lax.axis_index("c")
    base = wid * _RW

    def chunk_src(c):
        return md_hbm.at[pl.ds(base + c * _CH, _CH)]

    def make_rows2_body(b):
        def body(i, carry):
            out = list(carry)
            r0 = 2 * i
            for c in range(_G1):
                v0 = buf[b, r0, pl.ds(c * _L, _L)]
                v1 = buf[b, r0 + 1, pl.ds(c * _L, _L)]
                out[c] = (out[c] + v0) + v1
                out[_G1 + c] = (out[_G1 + c] + v0 * v0) + v1 * v1
            return tuple(out)
        return body

    def make_row_body(b):
        def body(r, carry):
            out = list(carry)
            for c in range(_G1):
                v = buf[b, r, pl.ds(c * _L, _L)]
                out[c] = out[c] + v
                out[_G1 + c] = out[_G1 + c] + v * v
            return tuple(out)
        return body

    for c in range(_NB - 1):
        pltpu.async_copy(chunk_src(c), buf.at[c], sems.at[c])

    def chunk_body(g, carry):
        b = lax.rem(g, _NB)
        pltpu.make_async_copy(chunk_src(g), buf.at[b], sems.at[b]).wait()

        @pl.when(g + _NB - 1 < _NCH)
        def _():
            b2 = lax.rem(g + _NB - 1, _NB)
            pltpu.async_copy(chunk_src(g + _NB - 1), buf.at[b2], sems.at[b2])

        return lax.fori_loop(0, _CH // 2, make_rows2_body(b), carry)

    acc = tuple(jnp.zeros((_L,), jnp.float32) for _ in range(2 * _G1))
    acc = lax.fori_loop(0, _NCH, chunk_body, acc)

    has_tail = wid < _NTAILW

    @pl.when(has_tail)
    def _():
        pltpu.sync_copy(md_hbm.at[pl.ds(_NW * _RW + 8 * wid, 8)],
                        buf.at[0, pl.ds(0, 8)])

    acc = lax.fori_loop(0, jnp.where(has_tail, 8, 0),
                        make_row_body(0), acc)

    for c in range(_G1):
        statbuf[0, pl.ds(c * _L, _L)] = acc[c]
        statbuf[0, pl.ds(_D1 + c * _L, _L)] = acc[_G1 + c]
    pltpu.sync_copy(statbuf, out_hbm.at[wid])


def _encoder_body(parts_ref, x_ref, w_ref, b_ref, out_ref):
    parts = parts_ref[:, 0, :]
    sums = jnp.sum(parts[:, :_D1], axis=0, keepdims=True)
    sumsq = jnp.sum(parts[:, _D1:], axis=0, keepdims=True)
    mean = sums / _N
    var = jnp.maximum((sumsq - sums * mean) / (_N - 1), 0.0)
    std = jnp.sqrt(var)
    new = (x_ref[...] - mean) / std
    new = jnp.where(std == 0.0, jnp.zeros_like(new), new)
    z = jnp.dot(new, w_ref[...], preferred_element_type=jnp.float32)
    out_ref[...] = jnp.tanh(z + b_ref[...])


_encoder = pl.pallas_call(
    _encoder_body,
    out_shape=jax.ShapeDtypeStruct((1, _D2), jnp.float32),
)


@functools.partial(
    pl.kernel,
    out_type=jax.ShapeDtypeStruct((_NW, 1, _L), jnp.float32),
    mesh=_mesh,
    scratch_types=[
        pltpu.VMEM((_NB, _CH, _D2), jnp.float32),
        pltpu.VMEM((_D2,), jnp.float32),
        pltpu.VMEM((1, _L), jnp.float32),
        pltpu.SemaphoreType.DMA((_NB,)),
    ],
)
def _dist_kernel(mem_hbm, e_hbm, out_hbm, buf, e_v, min_v, sems):
    wid = lax.axis_index("s") * _NC + lax.axis_index("c")
    base = wid * _RW

    pltpu.sync_copy(e_hbm.at[0], e_v)

    iota16 = lax.iota(jnp.int32, _L)
    _dnums = lax.GatherDimensionNumbers(
        offset_dims=(), collapsed_slice_dims=(0,), start_index_map=(0,))
    perms = [(iota16 ^ k).reshape(_L, 1) for k in (1, 2, 4, 8)]

    def lane_total(v):
        for perm in perms:
            v = v + lax.gather(v, perm, _dnums, slice_sizes=(1,),
                               mode=lax.GatherScatterMode.PROMISE_IN_BOUNDS)
        return v

    def make_block_body(b):
        def body(i, m):
            r0 = _BLK * i
            ch = [[None, None] for _ in range(_BLK)]
            for c in range(_G2):
                ev = e_v[pl.ds(c * _L, _L)]
                for r in range(_BLK):
                    d = jnp.abs(buf[b, r0 + r, pl.ds(c * _L, _L)] - ev)
                    j = c % 2
                    ch[r][j] = d if ch[r][j] is None else ch[r][j] + d
            for r in range(_BLK):
                m = jnp.minimum(m, lane_total(ch[r][0] + ch[r][1]))
            return m
        return body

    def make_row_body(b):
        def body(r, m):
            a0 = jnp.abs(buf[b, r, pl.ds(0, _L)] - e_v[pl.ds(0, _L)])
            a1 = jnp.abs(buf[b, r, pl.ds(_L, _L)] - e_v[pl.ds(_L, _L)])
            for c in range(2, _G2):
                d = jnp.abs(buf[b, r, pl.ds(c * _L, _L)]
                            - e_v[pl.ds(c * _L, _L)])
                if c % 2 == 0:
                    a0 = a0 + d
                else:
                    a1 = a1 + d
            return jnp.minimum(m, lane_total(a0 + a1))
        return body

    def chunk_src(c):
        return mem_hbm.at[pl.ds(base + c * _CH, _CH)]

    for c in range(_NB - 1):
        pltpu.async_copy(chunk_src(c), buf.at[c], sems.at[c])

    def chunk_body(g, m):
        b = lax.rem(g, _NB)
        pltpu.make_async_copy(chunk_src(g), buf.at[b], sems.at[b]).wait()

        @pl.when(g + _NB - 1 < _NCH)
        def _():
            b2 = lax.rem(g + _NB - 1, _NB)
            pltpu.async_copy(chunk_src(g + _NB - 1), buf.at[b2], sems.at[b2])

        return lax.fori_loop(0, _CH // _BLK, make_block_body(b), m)

    m = jnp.full((_L,), jnp.inf, jnp.float32)
    m = lax.fori_loop(0, _NCH, chunk_body, m)

    has_tail = wid < _NTAILW

    @pl.when(has_tail)
    def _():
        pltpu.sync_copy(mem_hbm.at[pl.ds(_NW * _RW + 8 * wid, 8)],
                        buf.at[0, pl.ds(0, 8)])

    m = lax.fori_loop(0, jnp.where(has_tail, 8, 0), make_row_body(0), m)

    min_v[...] = m.reshape(1, _L)
    pltpu.sync_copy(min_v, out_hbm.at[wid])


def kernel(x, memory, mem_data, W_enc, b_enc):
    parts = _stats_kernel(mem_data)
    e = _encoder(parts, x, W_enc, b_enc.reshape(1, _D2))
    mins = _dist_kernel(memory, e)
    return jnp.min(mins)
